# Initial kernel scaffold; baseline (speedup 1.0000x reference)
import jax
import jax.numpy as jnp
from jax.experimental import pallas as pl


def kernel(x, logit, label):
    raise NotImplementedError("write your pallas kernel here")


# speedup vs baseline: 1.2738x; 1.2738x over previous
import jax
import jax.numpy as jnp
from functools import partial
from jax.experimental import pallas as pl
from jax.experimental.pallas import tpu as pltpu

_NCLS = 19
_B = 4
_C = 256
_CT = 32
_HW = 16384
_NBIG = 175
_NSMALL = 75
_CHUNK = 2048
_NCH = _HW // _CHUNK


def _interp_kernel(x_ref, rm_ref, cm_ref, xi_ref, ssq_ref):
    ct = pl.program_id(1)
    xb = x_ref[0]
    tmp = jax.lax.dot_general(xb, cm_ref[...], (((2,), (1,)), ((), ())),
                              preferred_element_type=jnp.float32)
    rmb = jnp.broadcast_to(rm_ref[...][None], (_CT, 128, 64))
    xi = jax.lax.dot_general(rmb, tmp, (((2,), (1,)), ((0,), (0,))),
                             preferred_element_type=jnp.float32)
    xi_ref[0] = xi
    psum = jnp.sum(xi * xi, axis=0)

    @pl.when(ct == 0)
    def _():
        ssq_ref[0] = psum

    @pl.when(ct != 0)
    def _():
        ssq_ref[0] += psum


def _pm_kernel(lg_ref, pm_ref):
    l19 = lg_ref[0]
    m = jnp.max(l19, axis=0, keepdims=True)
    e = jnp.exp(l19 - m)
    p = e / jnp.sum(e, axis=0, keepdims=True)
    pe = jnp.exp(p)
    pmax = jnp.max(pe, axis=0, keepdims=True)
    iota = jax.lax.broadcasted_iota(jnp.int32, (_NCLS, _HW), 0)
    pm_ref[0] = jnp.min(jnp.where(pe == pmax, iota, _NCLS), axis=0,
                        keepdims=True)


def _sampler_kernel(noise_ref, pm_ref, lb_ref, ssq_ref, hist_ref,
                    vmax_ref, vidx_ref, *, count, gbase):
    j = pl.program_id(0)
    ch = pl.program_id(1)
    cls = j // 8
    gg = j % 2
    off = ch * _CHUNK
    pmc = pm_ref[0, :, pl.ds(off, _CHUNK)]
    lbc = lb_ref[0, :, pl.ds(off, _CHUNK)]
    peq = pmc == cls
    leq = lbc == cls
    if gbase == 0:
        m0 = peq & leq
        m1 = jnp.logical_and(~peq, ~leq)
    else:
        m0 = jnp.logical_and(peq, ~leq)
        m1 = jnp.logical_and(~peq, leq)
    lgv0 = jnp.where(m0, 0.0, -1e9).astype(jnp.float32)
    lgv1 = jnp.where(m1, 0.0, -1e9).astype(jnp.float32)
    lgv = jnp.where(gg == 0, lgv0, lgv1)
    vals = noise_ref[0] + lgv
    cmax = jnp.max(vals, axis=1, keepdims=True)
    iota = jax.lax.broadcasted_iota(jnp.int32, (count, _CHUNK), 1)
    ii = jnp.min(jnp.where(vals == cmax, iota, _HW), axis=1,
                 keepdims=True) + off

    @pl.when(ch == 0)
    def _():
        vmax_ref[...] = cmax
        vidx_ref[...] = ii

    @pl.when(ch != 0)
    def _():
        better = cmax > vmax_ref[...]
        vmax_ref[...] = jnp.where(better, cmax, vmax_ref[...])
        vidx_ref[...] = jnp.where(better, ii, vidx_ref[...])

    @pl.when(ch == _NCH - 1)
    def _():
        winners = vidx_ref[...]
        for cc in range(_NCH):
            it2 = jax.lax.broadcasted_iota(jnp.int32, (count, _CHUNK), 1)
            it2 = it2 + cc * _CHUNK
            cnt = jnp.sum(jnp.where(winners == it2, 1.0, 0.0), axis=0,
                          keepdims=True)
            w = 1.0 / (1e-6 + jnp.sqrt(ssq_ref[0, :, cc * _CHUNK:
                                               (cc + 1) * _CHUNK]))
            hist_ref[0, :, cc * _CHUNK:(cc + 1) * _CHUNK] = cnt * w


def _contract_kernel(xi_ref, hb_ref, hs_ref, out_ref, acc_ref):
    b = pl.program_id(0)
    ch = pl.program_id(1)
    xb = xi_ref[0]
    hb = hb_ref[:, 0, :, :]
    hs = hs_ref[:, 0, :, :]
    rhs = jnp.concatenate(
        [hb[:, 0, :], hb[:, 1, :], hs[:, 0, :], hs[:, 1, :]], axis=0)
    part = jax.lax.dot_general(xb, rhs, (((1,), (1,)), ((), ())),
                               preferred_element_type=jnp.float32)

    @pl.when(jnp.logical_and(b == 0, ch == 0))
    def _():
        acc_ref[...] = part

    @pl.when(jnp.logical_not(jnp.logical_and(b == 0, ch == 0)))
    def _():
        acc_ref[...] += part

    @pl.when(jnp.logical_and(b == _B - 1, ch == _NCH - 1))
    def _():
        s = acc_ref[...]
        d1 = s[:, 57:76] - s[:, 38:57]
        d2 = s[:, 0:19] - s[:, 19:38]
        tot = jnp.sum(d1 * d2, axis=(0, 1), keepdims=True)
        out_ref[...] = tot / (1200.0 * 700.0) / float(_NCLS)


def kernel(x, logit, label):
    x = x.astype(jnp.float32)
    logit = logit.astype(jnp.float32)

    rows = jnp.linspace(0.0, 63.0, 128)
    r0 = jnp.floor(rows).astype(jnp.int32)
    r1 = jnp.minimum(r0 + 1, 63)
    fr = (rows - r0.astype(jnp.float32)).astype(jnp.float32)
    eye = jnp.eye(64, dtype=jnp.float32)
    rmat = eye[r0] * (1.0 - fr)[:, None] + eye[r1] * fr[:, None]

    skey = jax.random.key(42)
    folds_big, folds_small = [], []
    for cls in range(_NCLS):
        for bi in range(_B):
            base = cls * 100 + bi * 10
            folds_big += [base + 0, base + 1]
            folds_small += [base + 2, base + 3]
    kb = jax.vmap(jax.random.fold_in, (None, 0))(skey, jnp.array(folds_big))
    ks = jax.vmap(jax.random.fold_in, (None, 0))(skey, jnp.array(folds_small))
    noise_big = jax.vmap(
        lambda k: jax.random.gumbel(k, (_NBIG, _HW), jnp.float32))(kb)
    noise_small = jax.vmap(
        lambda k: jax.random.gumbel(k, (_NSMALL, _HW), jnp.float32))(ks)

    xi, ssq = pl.pallas_call(
        _interp_kernel,
        grid=(_B, _C // _CT),
        in_specs=[
            pl.BlockSpec((1, _CT, 64, 64), lambda b, t: (b, t, 0, 0)),
            pl.BlockSpec((128, 64), lambda b, t: (0, 0)),
            pl.BlockSpec((128, 64), lambda b, t: (0, 0)),
        ],
        out_specs=[
            pl.BlockSpec((1, _CT, 128, 128), lambda b, t: (b, t, 0, 0)),
            pl.BlockSpec((1, 128, 128), lambda b, t: (b, 0, 0)),
        ],
        out_shape=[
            jax.ShapeDtypeStruct((_B, _C, 128, 128), jnp.float32),
            jax.ShapeDtypeStruct((_B, 128, 128), jnp.float32),
        ],
    )(x, rmat, rmat)

    pm = pl.pallas_call(
        _pm_kernel,
        grid=(_B,),
        in_specs=[pl.BlockSpec((1, _NCLS, _HW), lambda b: (b, 0, 0))],
        out_specs=pl.BlockSpec((1, 1, _HW), lambda b: (b, 0, 0)),
        out_shape=jax.ShapeDtypeStruct((_B, 1, _HW), jnp.int32),
    )(logit.reshape(_B, _NCLS, _HW))

    lb = label.astype(jnp.int32).reshape(_B, 1, _HW)
    ssq3 = ssq.reshape(_B, 1, _HW)
    ngroups = 2 * _B * _NCLS

    def run_sampler(noise, count, gbase):
        return pl.pallas_call(
            partial(_sampler_kernel, count=count, gbase=gbase),
            grid=(ngroups, _NCH),
            in_specs=[
                pl.BlockSpec((1, count, _CHUNK), lambda j, ch: (j, 0, ch)),
                pl.BlockSpec((1, 1, _HW), lambda j, ch: ((j // 2) % _B, 0, 0)),
                pl.BlockSpec((1, 1, _HW), lambda j, ch: ((j // 2) % _B, 0, 0)),
                pl.BlockSpec((1, 1, _HW), lambda j, ch: ((j // 2) % _B, 0, 0)),
            ],
            out_specs=pl.BlockSpec((1, 1, _HW), lambda j, ch: (j, 0, 0)),
            out_shape=jax.ShapeDtypeStruct((ngroups, 1, _HW), jnp.float32),
            scratch_shapes=[
                pltpu.VMEM((count, 1), jnp.float32),
                pltpu.VMEM((count, 1), jnp.int32),
            ],
        )(noise, pm, lb, ssq3)

    hist_big = run_sampler(noise_big, _NBIG, 0)
    hist_small = run_sampler(noise_small, _NSMALL, 2)

    hb4 = hist_big.reshape(_NCLS, _B, 2, _HW)
    hs4 = hist_small.reshape(_NCLS, _B, 2, _HW)

    out = pl.pallas_call(
        _contract_kernel,
        grid=(_B, _NCH),
        in_specs=[
            pl.BlockSpec((1, _C, _CHUNK), lambda b, ch: (b, 0, ch)),
            pl.BlockSpec((_NCLS, 1, 2, _CHUNK), lambda b, ch: (0, b, 0, ch)),
            pl.BlockSpec((_NCLS, 1, 2, _CHUNK), lambda b, ch: (0, b, 0, ch)),
        ],
        out_specs=pl.BlockSpec((1, 1), lambda b, ch: (0, 0)),
        out_shape=jax.ShapeDtypeStruct((1, 1), jnp.float32),
        scratch_shapes=[pltpu.VMEM((_C, 76), jnp.float32)],
    )(xi.reshape(_B, _C, _HW), hb4, hs4)

    return out[0, 0]
